# baseline (speedup 1.0000x reference)
import functools

import jax
import jax.numpy as jnp
from jax import lax
from jax.experimental import pallas as pl
from jax.experimental.pallas import tpu as pltpu
from jax.experimental.pallas import tpu_sc as plsc

_D = 128
_B = 16384
_NW = 16
_L = 16
_BPW = _B // _NW
_HW = _BPW // 2

_mesh = plsc.VectorSubcoreMesh(
    core_axis_name="c", subcore_axis_name="s", num_cores=1)


@functools.partial(
    pl.kernel,
    mesh=_mesh,
    out_type=jax.ShapeDtypeStruct((_B,), jnp.float32),
    scratch_types=[
        pltpu.VMEM((_BPW,), jnp.int32),
        pltpu.VMEM((_BPW,), jnp.int32),
        pltpu.VMEM((_BPW,), jnp.int32),
        pltpu.VMEM((_BPW,), jnp.float32),
        pltpu.SemaphoreType.DMA,
        pltpu.SemaphoreType.DMA,
        pltpu.SemaphoreType.DMA,
        pltpu.SemaphoreType.DMA,
    ],
)
def _gather_kernel(uid_hbm, iid_hbm, tab_hbm, out_hbm,
                   uid_v, iid_v, idx_v, val_v,
                   sem_in, sem_g1, sem_g2, sem_st):
    wid = lax.axis_index("s")
    base = wid * _BPW
    ld_u1 = pltpu.async_copy(uid_hbm.at[pl.ds(base, _HW)],
                             uid_v.at[pl.ds(0, _HW)], sem_in)
    ld_i1 = pltpu.async_copy(iid_hbm.at[pl.ds(base, _HW)],
                             iid_v.at[pl.ds(0, _HW)], sem_in)
    ld_u2 = pltpu.async_copy(uid_hbm.at[pl.ds(base + _HW, _HW)],
                             uid_v.at[pl.ds(_HW, _HW)], sem_in)
    ld_i2 = pltpu.async_copy(iid_hbm.at[pl.ds(base + _HW, _HW)],
                             iid_v.at[pl.ds(_HW, _HW)], sem_in)
    ld_u1.wait()
    ld_i1.wait()
    for k in range(_HW // _L):
        off = k * _L
        u = uid_v[pl.ds(off, _L)]
        it = iid_v[pl.ds(off, _L)]
        idx_v[pl.ds(off, _L)] = u * _D + it
    g1 = pltpu.async_copy(tab_hbm.at[idx_v.at[pl.ds(0, _HW)]],
                          val_v.at[pl.ds(0, _HW)], sem_g1)
    ld_u2.wait()
    ld_i2.wait()
    for k in range(_HW // _L):
        off = _HW + k * _L
        u = uid_v[pl.ds(off, _L)]
        it = iid_v[pl.ds(off, _L)]
        idx_v[pl.ds(off, _L)] = u * _D + it
    g2 = pltpu.async_copy(tab_hbm.at[idx_v.at[pl.ds(_HW, _HW)]],
                          val_v.at[pl.ds(_HW, _HW)], sem_g2)
    g1.wait()
    st1 = pltpu.async_copy(val_v.at[pl.ds(0, _HW)],
                           out_hbm.at[pl.ds(base, _HW)], sem_st)
    g2.wait()
    st2 = pltpu.async_copy(val_v.at[pl.ds(_HW, _HW)],
                           out_hbm.at[pl.ds(base + _HW, _HW)], sem_st)
    st1.wait()
    st2.wait()


def kernel(user_ids, item_ids, items_cnts):
    flat_table = items_cnts.reshape(-1)
    return _gather_kernel(user_ids.astype(jnp.int32),
                          item_ids.astype(jnp.int32),
                          flat_table)
